# Initial kernel scaffold; baseline (speedup 1.0000x reference)
import jax
import jax.numpy as jnp
from jax.experimental import pallas as pl


def kernel(x, edge_index, batch, atom_emb, W_in, b_in, W_out, b_out, W_mlp1, b_mlp1, W_mlp2, b_mlp2):
    raise NotImplementedError("write your pallas kernel here")


# speedup vs baseline: 17.2699x; 17.2699x over previous
import functools

import jax
import jax.numpy as jnp
from jax import lax
from jax.experimental import pallas as pl
from jax.experimental.pallas import tpu as pltpu
from jax.experimental.pallas import tpu_sc as plsc

N = 10000
E = 320000
F = 128
NUM_FEATS = 9
VOCAB = 119
NUM_GRAPHS = 64

NC = 2
NS = 16
NW = NC * NS
NPAD = 10240
EPT = E // NW
CHUNK = 125
NCHUNK = EPT // CHUNK
RPS = NPAD // NS
BLK = 256
NBLK = NPAD // BLK
DEGW = 8

@functools.cache
def _mesh():
    return plsc.VectorSubcoreMesh(core_axis_name="c", subcore_axis_name="s",
                                  num_cores=NC, num_subcores=NS)


def _deg_body(d_hbm, ones_hbm, z_hbm, out_hbm, didx_v, ones_v, acc_sh):
    cid = lax.axis_index("c")
    sid = lax.axis_index("s")
    wid = cid * NS + sid
    pltpu.sync_copy(z_hbm, acc_sh.at[pl.ds(sid * RPS, RPS)])
    pltpu.sync_copy(ones_hbm, ones_v)
    pltpu.sync_copy(d_hbm.at[wid], didx_v)
    plsc.subcore_barrier()

    def body(j, carry):
        pltpu.sync_copy(ones_v, acc_sh.at[didx_v.at[j]], add=True)
        return carry

    lax.fori_loop(0, NCHUNK, body, 0)
    plsc.subcore_barrier()
    pltpu.sync_copy(acc_sh.at[pl.ds(sid * RPS, RPS)],
                    out_hbm.at[cid, pl.ds(sid * RPS, RPS)])


@functools.cache
def _deg_call():
    return pl.kernel(
        _deg_body,
        out_type=jax.ShapeDtypeStruct((NC, NPAD, DEGW), jnp.float32),
        mesh=_mesh(),
        scratch_types=[
            pltpu.VMEM((NCHUNK, CHUNK), jnp.int32),
            pltpu.VMEM((CHUNK, DEGW), jnp.float32),
            pltpu.VMEM_SHARED((NPAD, DEGW), jnp.float32),
        ],
    )


def _scat_body(g_hbm, s_hbm, d_hbm, z_hbm, out_hbm,
               sidx_v, didx_v, rows_v, acc_sh, sem):
    cid = lax.axis_index("c")
    sid = lax.axis_index("s")
    wid = cid * NS + sid
    pltpu.sync_copy(z_hbm, acc_sh.at[pl.ds(sid * RPS, RPS)])
    pltpu.sync_copy(s_hbm.at[wid], sidx_v)
    pltpu.sync_copy(d_hbm.at[wid], didx_v)
    plsc.subcore_barrier()

    def body(j, carry):
        pltpu.async_copy(g_hbm.at[sidx_v.at[j]], rows_v, sem).wait()
        pltpu.sync_copy(rows_v, acc_sh.at[didx_v.at[j]], add=True)
        return carry

    lax.fori_loop(0, NCHUNK, body, 0)
    plsc.subcore_barrier()
    pltpu.sync_copy(acc_sh.at[pl.ds(sid * RPS, RPS)],
                    out_hbm.at[cid, pl.ds(sid * RPS, RPS)])


@functools.cache
def _scat_call():
    return pl.kernel(
        _scat_body,
        out_type=jax.ShapeDtypeStruct((NC, NPAD, F), jnp.float32),
        mesh=_mesh(),
        scratch_types=[
            pltpu.VMEM((NCHUNK, CHUNK), jnp.int32),
            pltpu.VMEM((NCHUNK, CHUNK), jnp.int32),
            pltpu.VMEM((CHUNK, F), jnp.float32),
            pltpu.VMEM_SHARED((NPAD, F), jnp.float32),
            pltpu.SemaphoreType.DMA,
        ],
    )


def _dinv_body(degp_ref, out_ref):
    p = degp_ref[...]
    out_ref[...] = lax.rsqrt(1.0 + p[0, :, 0:1] + p[1, :, 0:1])


def _dinv(degp):
    return pl.pallas_call(
        _dinv_body,
        grid=(NBLK,),
        in_specs=[pl.BlockSpec((NC, BLK, DEGW), lambda i: (0, i, 0))],
        out_specs=pl.BlockSpec((BLK, 1), lambda i: (i, 0)),
        out_shape=jax.ShapeDtypeStruct((NPAD, 1), jnp.float32),
    )(degp)


def _enc_body(x_ref, emb_ref, out_ref):
    acc = jnp.zeros((BLK, F), jnp.float32)
    for f in range(NUM_FEATS):
        col = x_ref[:, f:f + 1]
        oh = (lax.broadcasted_iota(jnp.int32, (BLK, VOCAB), 1) == col)
        acc += jnp.dot(oh.astype(jnp.float32), emb_ref[f],
                       preferred_element_type=jnp.float32)
    out_ref[...] = acc


def _encode(x_pad, atom_emb):
    return pl.pallas_call(
        _enc_body,
        grid=(NBLK,),
        in_specs=[
            pl.BlockSpec((BLK, NUM_FEATS), lambda i: (i, 0)),
            pl.BlockSpec((NUM_FEATS, VOCAB, F), lambda i: (0, 0, 0)),
        ],
        out_specs=pl.BlockSpec((BLK, F), lambda i: (i, 0)),
        out_shape=jax.ShapeDtypeStruct((NPAD, F), jnp.float32),
    )(x_pad, atom_emb)


def _mm_body(h_ref, w_ref, dinv_ref, out_ref):
    out_ref[...] = dinv_ref[...] * jnp.dot(
        h_ref[...], w_ref[...], preferred_element_type=jnp.float32)


def _scale_mm(h, w, dinv):
    return pl.pallas_call(
        _mm_body,
        grid=(NBLK,),
        in_specs=[
            pl.BlockSpec((BLK, F), lambda i: (i, 0)),
            pl.BlockSpec((F, F), lambda i: (0, 0)),
            pl.BlockSpec((BLK, 1), lambda i: (i, 0)),
        ],
        out_specs=pl.BlockSpec((BLK, F), lambda i: (i, 0)),
        out_shape=jax.ShapeDtypeStruct((NPAD, F), jnp.float32),
    )(h, w, dinv)


def _cmm_body(p_ref, g_ref, dinv_ref, b_ref, w_ref, out_ref):
    dinv = dinv_ref[...]
    h = jnp.maximum(dinv * (p_ref[0] + p_ref[1] + g_ref[...]) + b_ref[...], 0.0)
    out_ref[...] = dinv * jnp.dot(h, w_ref[...],
                                  preferred_element_type=jnp.float32)


def _combine_scale_mm(p, g, dinv, b, w):
    return pl.pallas_call(
        _cmm_body,
        grid=(NBLK,),
        in_specs=[
            pl.BlockSpec((NC, BLK, F), lambda i: (0, i, 0)),
            pl.BlockSpec((BLK, F), lambda i: (i, 0)),
            pl.BlockSpec((BLK, 1), lambda i: (i, 0)),
            pl.BlockSpec((1, F), lambda i: (0, 0)),
            pl.BlockSpec((F, F), lambda i: (0, 0)),
        ],
        out_specs=pl.BlockSpec((BLK, F), lambda i: (i, 0)),
        out_shape=jax.ShapeDtypeStruct((NPAD, F), jnp.float32),
    )(p, g, dinv, b, w)


def _pool_body(p_ref, g_ref, dinv_ref, b_ref, batch_ref, w1_ref, b1_ref,
               w2_ref, b2_ref, out_ref, sums, counts):
    i = pl.program_id(0)

    @pl.when(i == 0)
    def _():
        sums[...] = jnp.zeros((NUM_GRAPHS, F), jnp.float32)
        counts[...] = jnp.zeros((NUM_GRAPHS, 1), jnp.float32)

    dinv = dinv_ref[...]
    h = jnp.maximum(dinv * (p_ref[0] + p_ref[1] + g_ref[...]) + b_ref[...], 0.0)
    pt = (lax.broadcasted_iota(jnp.int32, (BLK, NUM_GRAPHS), 1)
          == batch_ref[...]).astype(jnp.float32)
    dn = (((0,), (0,)), ((), ()))
    sums[...] += lax.dot_general(pt, h, dn, preferred_element_type=jnp.float32)
    counts[...] += lax.dot_general(pt, jnp.ones((BLK, 1), jnp.float32), dn,
                                   preferred_element_type=jnp.float32)

    @pl.when(i == pl.num_programs(0) - 1)
    def _():
        pooled = sums[...] / jnp.maximum(counts[...], 1.0)
        t = jnp.maximum(
            jnp.dot(pooled, w1_ref[...], preferred_element_type=jnp.float32)
            + b1_ref[...], 0.0)
        out_ref[...] = jnp.dot(t, w2_ref[...],
                               preferred_element_type=jnp.float32) + b2_ref[...]


def _pool_mlp(p, g, dinv, b, batch2d, w1, b1, w2, b2):
    return pl.pallas_call(
        _pool_body,
        grid=(NBLK,),
        in_specs=[
            pl.BlockSpec((NC, BLK, F), lambda i: (0, i, 0)),
            pl.BlockSpec((BLK, F), lambda i: (i, 0)),
            pl.BlockSpec((BLK, 1), lambda i: (i, 0)),
            pl.BlockSpec((1, F), lambda i: (0, 0)),
            pl.BlockSpec((BLK, 1), lambda i: (i, 0)),
            pl.BlockSpec((F, F), lambda i: (0, 0)),
            pl.BlockSpec((1, F), lambda i: (0, 0)),
            pl.BlockSpec((F, F), lambda i: (0, 0)),
            pl.BlockSpec((1, F), lambda i: (0, 0)),
        ],
        out_specs=pl.BlockSpec((NUM_GRAPHS, F), lambda i: (0, 0)),
        out_shape=jax.ShapeDtypeStruct((NUM_GRAPHS, F), jnp.float32),
        scratch_shapes=[
            pltpu.VMEM((NUM_GRAPHS, F), jnp.float32),
            pltpu.VMEM((NUM_GRAPHS, 1), jnp.float32),
        ],
    )(p, g, dinv, b, batch2d, w1, b1, w2, b2)


def kernel(x, edge_index, batch, atom_emb, W_in, b_in, W_out, b_out,
           W_mlp1, b_mlp1, W_mlp2, b_mlp2):
    x_pad = jnp.pad(x, ((0, NPAD - N), (0, 0)))
    batch2d = jnp.pad(batch, (0, NPAD - N),
                      constant_values=NUM_GRAPHS).reshape(NPAD, 1)
    s_slab = edge_index[0].reshape(NW, NCHUNK, CHUNK)
    d_slab = edge_index[1].reshape(NW, NCHUNK, CHUNK)
    ones_deg = jnp.ones((CHUNK, DEGW), jnp.float32)
    zeros_deg = jnp.zeros((RPS, DEGW), jnp.float32)
    zeros_rows = jnp.zeros((RPS, F), jnp.float32)
    b_in2 = b_in.reshape(1, F)
    b_out2 = b_out.reshape(1, F)
    b_mlp12 = b_mlp1.reshape(1, F)
    b_mlp22 = b_mlp2.reshape(1, F)

    degp = _deg_call()(d_slab, ones_deg, zeros_deg)
    dinv = _dinv(degp)

    h0 = _encode(x_pad, atom_emb)

    g1 = _scale_mm(h0, W_in, dinv)
    p1 = _scat_call()(g1, s_slab, d_slab, zeros_rows)

    g2 = _combine_scale_mm(p1, g1, dinv, b_in2, W_out)
    p2 = _scat_call()(g2, s_slab, d_slab, zeros_rows)

    return _pool_mlp(p2, g2, dinv, b_out2, batch2d, W_mlp1, b_mlp12,
                     W_mlp2, b_mlp22)
